# baseline (speedup 1.0000x reference)
import functools

import jax
import jax.numpy as jnp
from jax import lax
from jax.experimental import pallas as pl
from jax.experimental.pallas import tpu as pltpu
from jax.experimental.pallas import tpu_sc as plsc

D = 768
_NV = D // 16
EPS = 1e-12

_info = plsc.get_sparse_core_info()
_NC, _NS = _info.num_cores, _info.num_subcores
_NW = _NC * _NS

N_TOK = 128 * 512
_PER_W = N_TOK // _NW
_CH = 32
_NCHUNK = _PER_W // _CH
_NGRP = _NCHUNK // 2

_mesh = plsc.VectorSubcoreMesh(core_axis_name="c", subcore_axis_name="s")


@functools.partial(
    pl.kernel,
    mesh=_mesh,
    out_type=jax.ShapeDtypeStruct((N_TOK, D), jnp.float32),
    scratch_types=[
        pltpu.VMEM((_PER_W,), jnp.int32),
        pltpu.VMEM((_CH, D), jnp.float32),
        pltpu.VMEM((_CH, D), jnp.float32),
        pltpu.VMEM((_CH, D), jnp.float32),
        pltpu.VMEM((_CH, D), jnp.float32),
        pltpu.VMEM((D,), jnp.float32),
        pltpu.VMEM((D,), jnp.float32),
        pltpu.SemaphoreType.DMA,
        pltpu.SemaphoreType.DMA,
        pltpu.SemaphoreType.DMA,
        pltpu.SemaphoreType.DMA,
    ],
)
def _sc_embed(table_hbm, idx_hbm, gamma_hbm, beta_hbm, out_hbm,
              idx_v, g0, g1, s0, s1, gv, bv, sg0, sg1, ss0, ss1):
    wid = lax.axis_index("s") * _NC + lax.axis_index("c")
    base = wid * _PER_W
    pltpu.sync_copy(idx_hbm.at[pl.ds(base, _PER_W)], idx_v)
    pltpu.sync_copy(gamma_hbm, gv)
    pltpu.sync_copy(beta_hbm, bv)

    gbufs = (g0, g1)
    sbufs = (s0, s1)
    sgs = (sg0, sg1)
    sss = (ss0, ss1)

    def issue_gather(c, b):
        pltpu.async_copy(table_hbm.at[idx_v.at[pl.ds(c * _CH, _CH)]],
                         gbufs[b], sgs[b])

    def wait_gather(c, b):
        pltpu.make_async_copy(table_hbm.at[idx_v.at[pl.ds(c * _CH, _CH)]],
                              gbufs[b], sgs[b]).wait()

    def issue_scatter(c, b):
        pltpu.async_copy(sbufs[b], out_hbm.at[pl.ds(base + c * _CH, _CH)],
                         sss[b])

    def wait_scatter(c, b):
        pltpu.make_async_copy(sbufs[b], out_hbm.at[pl.ds(base + c * _CH, _CH)],
                              sss[b]).wait()

    def ln_chunk(gb, sb):
        def row_body(r, carry):
            acc = gb[r, pl.ds(0, 16)]
            acc2 = acc * acc
            for j in range(1, _NV):
                x = gb[r, pl.ds(16 * j, 16)]
                acc = acc + x
                acc2 = acc2 + x * x
            dnums = lax.GatherDimensionNumbers(
                offset_dims=(), collapsed_slice_dims=(0,),
                start_index_map=(0,))

            def shuffle(x, perm):
                return lax.gather(
                    x, perm[:, None], dnums, slice_sizes=(1,),
                    mode=lax.GatherScatterMode.PROMISE_IN_BOUNDS)

            for k in (8, 4, 2, 1):
                perm = lax.iota(jnp.int32, 16) ^ k
                acc = acc + shuffle(acc, perm)
                acc2 = acc2 + shuffle(acc2, perm)
            mu_v = acc * jnp.float32(1.0 / D)
            v = acc2 * jnp.float32(1.0 / D) - mu_v * mu_v + jnp.float32(EPS)
            i = lax.bitcast_convert_type(v, jnp.int32)
            i = jnp.int32(0x5F3759DF) - lax.shift_right_logical(i, 1)
            y = lax.bitcast_convert_type(i, jnp.float32)
            half = v * jnp.float32(0.5)
            for _ in range(4):
                y = y * (jnp.float32(1.5) - half * y * y)
            for j in range(_NV):
                x = gb[r, pl.ds(16 * j, 16)]
                g = gv[pl.ds(16 * j, 16)]
                bb = bv[pl.ds(16 * j, 16)]
                sb[r, pl.ds(16 * j, 16)] = (x - mu_v) * y * g + bb
            return carry

        lax.fori_loop(0, _CH, row_body, 0, unroll=False)

    issue_gather(0, 0)
    issue_gather(1, 1)

    def group(t, carry):
        for b in range(2):
            c = 2 * t + b
            wait_gather(c, b)

            @pl.when(t >= 1)
            def _():
                wait_scatter(c - 2, b)

            ln_chunk(gbufs[b], sbufs[b])

            @pl.when(t <= _NGRP - 2)
            def _():
                issue_gather(c + 2, b)

            issue_scatter(c, b)
        return carry

    lax.fori_loop(0, _NGRP, group, 0, unroll=False)
    wait_scatter(_NCHUNK - 2, 0)
    wait_scatter(_NCHUNK - 1, 1)


def kernel(input_ids, token_type_ids, position_ids, word_table, pos_table,
           tt_table, ln_gamma, ln_beta):
    del token_type_ids, position_ids, pos_table, tt_table
    ids_flat = input_ids.reshape(N_TOK).astype(jnp.int32)
    out = _sc_embed(word_table, ids_flat, ln_gamma, ln_beta)
    B, S = input_ids.shape
    return out.reshape(B, S, D)


# speedup vs baseline: 3.5005x; 3.5005x over previous
import functools

import jax
import jax.numpy as jnp
from jax import lax
from jax.experimental import pallas as pl
from jax.experimental.pallas import tpu as pltpu
from jax.experimental.pallas import tpu_sc as plsc

VOCAB = 30522
D = 768
EPS = 1e-12


_ROWS_BLK = 2048


def _ln_body(x_ref, g_ref, b_ref, o_ref):
    x = x_ref[...]
    mu = jnp.mean(x, axis=-1, keepdims=True)
    xc = x - mu
    var = jnp.mean(xc * xc, axis=-1, keepdims=True)
    o_ref[...] = (xc * lax.rsqrt(var + EPS)) * g_ref[...] + b_ref[...]


def _normalize_table(word_table, ln_gamma, ln_beta):
    n_blocks = pl.cdiv(VOCAB, _ROWS_BLK)
    return pl.pallas_call(
        _ln_body,
        grid=(n_blocks,),
        in_specs=[
            pl.BlockSpec((_ROWS_BLK, D), lambda i: (i, 0)),
            pl.BlockSpec((1, D), lambda i: (0, 0)),
            pl.BlockSpec((1, D), lambda i: (0, 0)),
        ],
        out_specs=pl.BlockSpec((_ROWS_BLK, D), lambda i: (i, 0)),
        out_shape=jax.ShapeDtypeStruct((VOCAB, D), jnp.float32),
    )(word_table, ln_gamma.reshape(1, D), ln_beta.reshape(1, D))


_info = plsc.get_sparse_core_info()
_NC, _NS = _info.num_cores, _info.num_subcores
_NW = _NC * _NS

N_TOK = 128 * 512
_PER_W = N_TOK // _NW
_CH = 32
_NBUF = 4
_NCHUNK = _PER_W // _CH
_NGRP = _NCHUNK // _NBUF

_mesh = plsc.VectorSubcoreMesh(core_axis_name="c", subcore_axis_name="s")


@functools.partial(
    pl.kernel,
    mesh=_mesh,
    out_type=jax.ShapeDtypeStruct((N_TOK, D), jnp.float32),
    scratch_types=[
        pltpu.VMEM((_PER_W,), jnp.int32),
        pltpu.VMEM((_CH, D), jnp.float32),
        pltpu.VMEM((_CH, D), jnp.float32),
        pltpu.VMEM((_CH, D), jnp.float32),
        pltpu.VMEM((_CH, D), jnp.float32),
        pltpu.SemaphoreType.DMA,
        pltpu.SemaphoreType.DMA,
        pltpu.SemaphoreType.DMA,
        pltpu.SemaphoreType.DMA,
        pltpu.SemaphoreType.DMA,
        pltpu.SemaphoreType.DMA,
        pltpu.SemaphoreType.DMA,
        pltpu.SemaphoreType.DMA,
    ],
)
def _sc_gather(table_hbm, idx_hbm, out_hbm, idx_v,
               b0, b1, b2, b3, sg0, sg1, sg2, sg3, ss0, ss1, ss2, ss3):
    wid = lax.axis_index("s") * _NC + lax.axis_index("c")
    base = wid * _PER_W
    pltpu.sync_copy(idx_hbm.at[pl.ds(base, _PER_W)], idx_v)

    bufs = (b0, b1, b2, b3)
    sgs = (sg0, sg1, sg2, sg3)
    sss = (ss0, ss1, ss2, ss3)

    def issue_gather(c, b):
        pltpu.async_copy(table_hbm.at[idx_v.at[pl.ds(c * _CH, _CH)]],
                         bufs[b], sgs[b])

    def wait_gather(c, b):
        pltpu.make_async_copy(table_hbm.at[idx_v.at[pl.ds(c * _CH, _CH)]],
                              bufs[b], sgs[b]).wait()

    def issue_scatter(c, b):
        pltpu.async_copy(bufs[b], out_hbm.at[pl.ds(base + c * _CH, _CH)],
                         sss[b])

    def wait_scatter(c, b):
        pltpu.make_async_copy(bufs[b], out_hbm.at[pl.ds(base + c * _CH, _CH)],
                              sss[b]).wait()

    issue_gather(0, 0)
    issue_gather(1, 1)

    def group(t, carry):
        for b in range(_NBUF):
            c = t * _NBUF + b
            wait_gather(c, b)
            issue_scatter(c, b)
            b2 = (b + 2) % _NBUF

            @pl.when(jnp.logical_and(c + 2 < _NCHUNK, c - 2 >= 0))
            def _():
                wait_scatter(c - 2, b2)
                issue_gather(c + 2, b2)

            @pl.when(c - 2 < 0)
            def _():
                issue_gather(c + 2, b2)
        return carry

    lax.fori_loop(0, _NGRP, group, 0, unroll=False)
    wait_scatter(_NCHUNK - 2, 2)
    wait_scatter(_NCHUNK - 1, 3)


def kernel(input_ids, token_type_ids, position_ids, word_table, pos_table,
           tt_table, ln_gamma, ln_beta):
    del token_type_ids, position_ids, pos_table, tt_table
    normed = _normalize_table(word_table, ln_gamma, ln_beta)
    ids_flat = input_ids.reshape(N_TOK).astype(jnp.int32)
    out = _sc_gather(normed, ids_flat)
    B, S = input_ids.shape
    return out.reshape(B, S, D)
